# baseline (speedup 1.0000x reference)
import functools

import jax
import jax.numpy as jnp
from jax.experimental import pallas as pl
from jax.experimental.pallas import tpu as pltpu

_H = 2048
_E = 64
_TOP_K = 8
_N_GROUP = 8
_GROUP_SIZE = _E // _N_GROUP
_TOPK_GROUP = 4
_SCALE = 2.5
_NEG = float("-inf")


def _shift_up_rows(x, k):
    return pltpu.roll(x, _E - k, 0)


def _gate_kernel(x_ref, w_ref, b_ref, idx_ref, wgt_ref):
    x = x_ref[...]
    w = w_ref[...]
    logits = jnp.dot(x, w, preferred_element_type=jnp.float32)
    sT = jax.nn.sigmoid(logits.T) + b_ref[...]
    t = sT.shape[1]

    row = jax.lax.broadcasted_iota(jnp.int32, (_E, t), 0)
    row_f = row.astype(jnp.float32)
    grp_f = (row // _GROUP_SIZE).astype(jnp.float32)

    p1 = _shift_up_rows(sT, 1)
    b1 = jnp.maximum(sT, p1)
    b2 = jnp.minimum(sT, p1)
    for k in (2, 4):
        p1 = _shift_up_rows(b1, k)
        p2 = _shift_up_rows(b2, k)
        lo = jnp.minimum(b1, p1)
        b1 = jnp.maximum(b1, p1)
        b2 = jnp.maximum(lo, jnp.maximum(b2, p2))
    gs = jnp.where((row & (_GROUP_SIZE - 1)) == 0, b1 + b2, _NEG)

    smask = jnp.zeros((_E, t), jnp.bool_)
    for _ in range(_TOPK_GROUP):
        mx = jnp.max(gs, axis=0, keepdims=True)
        sel = jnp.min(jnp.where(gs >= mx, grp_f, 8.0), axis=0,
                      keepdims=True)
        hit = grp_f == sel
        smask = jnp.logical_or(smask, hit)
        gs = jnp.where(hit, _NEG, gs)

    tmp = jnp.where(smask, sT, 0.0)
    krow = jax.lax.broadcasted_iota(jnp.int32, (_TOP_K, t), 0)
    acc_i = jnp.zeros((_TOP_K, t), jnp.float32)
    acc_w = jnp.zeros((_TOP_K, t), jnp.float32)
    for kk in range(_TOP_K):
        mx = jnp.max(tmp, axis=0, keepdims=True)
        sel = jnp.min(jnp.where(tmp >= mx, row_f, 64.0), axis=0,
                      keepdims=True)
        hit = row_f == sel
        wv = jnp.max(jnp.where(hit, sT, _NEG), axis=0, keepdims=True)
        col = krow == kk
        acc_i = jnp.where(col, sel, acc_i)
        acc_w = jnp.where(col, wv, acc_w)
        tmp = jnp.where(hit, _NEG, tmp)

    denom = jnp.sum(acc_w, axis=0, keepdims=True) + 1e-20
    acc_w = acc_w * (_SCALE / denom)

    idx_ref[...] = acc_i.astype(jnp.int32)
    wgt_ref[...] = acc_w


@functools.partial(jax.jit, static_argnames=("interpret",))
def _gate(x2, weight, bias_col, interpret=False):
    n = x2.shape[0]
    t = 2048
    grid = (n // t,)
    return pl.pallas_call(
        _gate_kernel,
        grid=grid,
        in_specs=[
            pl.BlockSpec((t, _H), lambda i: (i, 0)),
            pl.BlockSpec((_H, _E), lambda i: (0, 0)),
            pl.BlockSpec((_E, 1), lambda i: (0, 0)),
        ],
        out_specs=[
            pl.BlockSpec((_TOP_K, t), lambda i: (0, i)),
            pl.BlockSpec((_TOP_K, t), lambda i: (0, i)),
        ],
        out_shape=[
            jax.ShapeDtypeStruct((_TOP_K, n), jnp.int32),
            jax.ShapeDtypeStruct((_TOP_K, n), jnp.float32),
        ],
        interpret=interpret,
    )(x2, weight, bias_col)


def kernel(hidden_states, weight, e_score_correction_bias):
    bsz, seq_len, h = hidden_states.shape
    x2 = hidden_states.reshape(bsz * seq_len, h)
    bias_col = e_score_correction_bias.reshape(_E, 1)
    idx_t, wgt_t = _gate(x2, weight, bias_col)
    return (idx_t.T, wgt_t.T)
